# baseline (speedup 1.0000x reference)
import functools

import jax
import jax.numpy as jnp
from jax import lax
from jax.experimental import pallas as pl
from jax.experimental.pallas import tpu as pltpu
from jax.experimental.pallas import tpu_sc as plsc

_N = 10000
_E = 320000
_F = 128
_H = 128
_C = 64

_NP = 10112

_NC = 2
_NS = 16
_NW = _NC * _NS
_EPW = _E // _NW
_K = 125
_NCHUNK = _EPW // _K
_RPT = _NP // _NS


def _make_agg(D, with_count, _NBUF, dtype=jnp.float32):
  mesh = plsc.VectorSubcoreMesh(
      core_axis_name="c", subcore_axis_name="s",
      num_cores=_NC, num_subcores=_NS)

  out_type = [jax.ShapeDtypeStruct((_NC, _NP, D), dtype)]
  scratch = [
      pltpu.VMEM((_NCHUNK, _K), jnp.int32),
      pltpu.VMEM((_NCHUNK, _K), jnp.int32),
  ] + [pltpu.VMEM((_K, D), dtype) for _ in range(_NBUF)] + [
      pltpu.VMEM_SHARED((_NP, D), dtype),
  ] + [pltpu.SemaphoreType.DMA for _ in range(_NBUF)]
  if with_count:
    out_type.append(jax.ShapeDtypeStruct((_NC, _NP, 16), jnp.float32))
    scratch += [
        pltpu.VMEM((_K, 16), jnp.float32),
        pltpu.VMEM((_RPT, 16), jnp.float32),
        pltpu.VMEM_SHARED((_NP, 16), jnp.float32),
    ]

  def body(table, e3, *rest):
    if with_count:
      (out_p, out_c, srcs_v, dsts_v, *rb) = rest
    else:
      (out_p, srcs_v, dsts_v, *rb) = rest
    rows = rb[:_NBUF]
    acc_sh = rb[_NBUF]
    gsems = rb[_NBUF + 1:2 * _NBUF + 1]
    if with_count:
      ones_v, zc_v, cnt_sh = rb[2 * _NBUF + 1:]
    cid = lax.axis_index("c")
    sid = lax.axis_index("s")
    wid = sid * _NC + cid
    row0 = sid * _RPT

    pltpu.sync_copy(e3.at[0, pl.ds(wid * _NCHUNK, _NCHUNK)], srcs_v)
    pltpu.sync_copy(e3.at[1, pl.ds(wid * _NCHUNK, _NCHUNK)], dsts_v)

    zlanes = 16 * 4 // jnp.dtype(dtype).itemsize
    def zfill(i, _):
      for c in range(D // zlanes):
        rows[0][i, pl.ds(c * zlanes, zlanes)] = jnp.zeros((zlanes,), dtype)
      return 0
    lax.fori_loop(0, _K, zfill, 0)
    for t in range(_RPT // _K):
      pltpu.sync_copy(rows[0], acc_sh.at[pl.ds(row0 + t * _K, _K)])
    rem = _RPT % _K
    if rem:
      pltpu.sync_copy(rows[0].at[pl.ds(0, rem)],
                      acc_sh.at[pl.ds(row0 + (_RPT // _K) * _K, rem)])

    if with_count:
      def ofill(i, _):
        ones_v[i, pl.ds(0, 16)] = jnp.ones((16,), jnp.float32)
        return 0
      lax.fori_loop(0, _K, ofill, 0)
      def zcfill(i, _):
        zc_v[i, pl.ds(0, 16)] = jnp.zeros((16,), jnp.float32)
        return 0
      lax.fori_loop(0, _RPT, zcfill, 0)
      pltpu.sync_copy(zc_v, cnt_sh.at[pl.ds(row0, _RPT)])

    plsc.subcore_barrier()

    for b in range(_NBUF):
      pltpu.async_copy(table.at[srcs_v.at[b]], rows[b], gsems[b])

    @pl.loop(0, _NCHUNK - _NBUF, step=_NBUF)
    def _steady(g):
      for b in range(_NBUF):
        j = g + b
        pltpu.make_async_copy(table.at[srcs_v.at[b]], rows[b], gsems[b]).wait()
        pltpu.sync_copy(rows[b], acc_sh.at[dsts_v.at[j]], add=True)
        if with_count:
          pltpu.sync_copy(ones_v, cnt_sh.at[dsts_v.at[j]], add=True)
        pltpu.async_copy(table.at[srcs_v.at[j + _NBUF]], rows[b], gsems[b])

    for b in range(_NBUF):
      j = _NCHUNK - _NBUF + b
      pltpu.make_async_copy(table.at[srcs_v.at[b]], rows[b], gsems[b]).wait()
      pltpu.sync_copy(rows[b], acc_sh.at[dsts_v.at[j]], add=True)
      if with_count:
        pltpu.sync_copy(ones_v, cnt_sh.at[dsts_v.at[j]], add=True)

    plsc.subcore_barrier()

    pltpu.sync_copy(acc_sh.at[pl.ds(row0, _RPT)],
                    out_p.at[cid, pl.ds(row0, _RPT)])
    if with_count:
      pltpu.sync_copy(cnt_sh.at[pl.ds(row0, _RPT)],
                      out_c.at[cid, pl.ds(row0, _RPT)])

  return pl.kernel(
      body, out_type=out_type, mesh=mesh, scratch_types=scratch,
      compiler_params=pltpu.CompilerParams(use_tc_tiling_on_sc=False))


_agg_c = _make_agg(128, with_count=True, _NBUF=4, dtype=jnp.bfloat16)
_agg_b = _make_agg(64, with_count=False, _NBUF=8, dtype=jnp.bfloat16)


_RB = 5056
_GRID = _NP // _RB


def _tc1_body(x_ref, wl_ref, wr_ref, a1_ref, r1_ref):
  x = x_ref[...]
  a1 = jnp.dot(x, wl_ref[...], preferred_element_type=jnp.float32)
  a1_ref[...] = a1.astype(jnp.bfloat16)
  r1_ref[...] = jnp.dot(x, wr_ref[...], preferred_element_type=jnp.float32)


_tc1 = pl.pallas_call(
    _tc1_body,
    grid=(_GRID,),
    in_specs=[
        pl.BlockSpec((_RB, _F), lambda i: (i, 0)),
        pl.BlockSpec((_F, _H), lambda i: (0, 0)),
        pl.BlockSpec((_F, _H), lambda i: (0, 0)),
    ],
    out_specs=[
        pl.BlockSpec((_RB, _H), lambda i: (i, 0)),
        pl.BlockSpec((_RB, _H), lambda i: (i, 0)),
    ],
    out_shape=[
        jax.ShapeDtypeStruct((_NP, _H), jnp.bfloat16),
        jax.ShapeDtypeStruct((_NP, _H), jnp.float32),
    ],
)


def _tc2_body(p_ref, c_ref, r1_ref, b1_ref, wl_ref, wr_ref,
              a2_ref, r2_ref):
  s1 = p_ref[0].astype(jnp.float32) + p_ref[1].astype(jnp.float32)
  cnt = jnp.max(c_ref[0] + c_ref[1], axis=1, keepdims=True)
  mean = s1 / jnp.maximum(cnt, 1.0)
  h = jnp.maximum(mean + r1_ref[...] + b1_ref[...], 0.0)
  a2 = jnp.dot(h, wl_ref[...], preferred_element_type=jnp.float32)
  a2_ref[...] = a2.astype(jnp.bfloat16)
  r2_ref[...] = jnp.dot(h, wr_ref[...], preferred_element_type=jnp.float32)


_tc2 = pl.pallas_call(
    _tc2_body,
    grid=(_GRID,),
    in_specs=[
        pl.BlockSpec((_NC, _RB, _H), lambda i: (0, i, 0)),
        pl.BlockSpec((_NC, _RB, 16), lambda i: (0, i, 0)),
        pl.BlockSpec((_RB, _H), lambda i: (i, 0)),
        pl.BlockSpec((1, _H), lambda i: (0, 0)),
        pl.BlockSpec((_H, _C), lambda i: (0, 0)),
        pl.BlockSpec((_H, _C), lambda i: (0, 0)),
    ],
    out_specs=[
        pl.BlockSpec((_RB, _C), lambda i: (i, 0)),
        pl.BlockSpec((_RB, _C), lambda i: (i, 0)),
    ],
    out_shape=[
        jax.ShapeDtypeStruct((_NP, _C), jnp.bfloat16),
        jax.ShapeDtypeStruct((_NP, _C), jnp.float32),
    ],
)


def _tc3_body(q_ref, c_ref, r2_ref, b2_ref, out_ref):
  s2 = q_ref[0].astype(jnp.float32) + q_ref[1].astype(jnp.float32)
  cnt = jnp.max(c_ref[0] + c_ref[1], axis=1, keepdims=True)
  o = s2 / jnp.maximum(cnt, 1.0) + r2_ref[...] + b2_ref[...]
  m = jnp.max(o, axis=1, keepdims=True)
  lse = jnp.log(jnp.sum(jnp.exp(o - m), axis=1, keepdims=True)) + m
  out_ref[...] = o - lse


_tc3 = pl.pallas_call(
    _tc3_body,
    grid=(_GRID,),
    in_specs=[
        pl.BlockSpec((_NC, _RB, _C), lambda i: (0, i, 0)),
        pl.BlockSpec((_NC, _RB, 16), lambda i: (0, i, 0)),
        pl.BlockSpec((_RB, _C), lambda i: (i, 0)),
        pl.BlockSpec((1, _C), lambda i: (0, 0)),
    ],
    out_specs=pl.BlockSpec((_RB, _C), lambda i: (i, 0)),
    out_shape=jax.ShapeDtypeStruct((_NP, _C), jnp.float32),
)


@jax.jit
def kernel(x, edge_index, W1_l, W1_r, b1, W2_l, W2_r, b2):
  e3 = edge_index.reshape(2, _E // _K, _K)
  x_p = jnp.pad(x, ((0, _NP - _N), (0, 0)))
  a1, r1 = _tc1(x_p, W1_l, W1_r)
  p, c = _agg_c(a1, e3)
  a2, r2 = _tc2(p, c, r1, b1.reshape(1, _H), W2_l, W2_r)
  (q,) = _agg_b(a2, e3)
  out = _tc3(q, c, r2, b2.reshape(1, _C))
  return out[:_N]


# speedup vs baseline: 1.0138x; 1.0138x over previous
import functools

import jax
import jax.numpy as jnp
from jax import lax
from jax.experimental import pallas as pl
from jax.experimental.pallas import tpu as pltpu
from jax.experimental.pallas import tpu_sc as plsc

_N = 10000
_E = 320000
_F = 128
_H = 128
_C = 64

_NP = 10112

_NC = 2
_NS = 16
_NW = _NC * _NS
_EPW = _E // _NW
_K = 125
_NCHUNK = _EPW // _K
_RPT = _NP // _NS


def _make_agg(D, with_count, _NBUF, dtype=jnp.float32):
  mesh = plsc.VectorSubcoreMesh(
      core_axis_name="c", subcore_axis_name="s",
      num_cores=_NC, num_subcores=_NS)

  out_type = [jax.ShapeDtypeStruct((_NC, _NP, D), dtype)]
  scratch = [
      pltpu.VMEM((_NCHUNK, _K), jnp.int32),
      pltpu.VMEM((_NCHUNK, _K), jnp.int32),
  ] + [pltpu.VMEM((_K, D), dtype) for _ in range(_NBUF)] + [
      pltpu.VMEM_SHARED((_NP, D), dtype),
  ] + [pltpu.SemaphoreType.DMA for _ in range(_NBUF)]
  if with_count:
    out_type.append(jax.ShapeDtypeStruct((_NC, _NP, 16), jnp.float32))
    scratch += [
        pltpu.VMEM((_K, 16), jnp.float32),
        pltpu.VMEM((_RPT, 16), jnp.float32),
        pltpu.VMEM_SHARED((_NP, 16), jnp.float32),
    ]

  def body(table, e3, *rest):
    if with_count:
      (out_p, out_c, srcs_v, dsts_v, *rb) = rest
    else:
      (out_p, srcs_v, dsts_v, *rb) = rest
    rows = rb[:_NBUF]
    acc_sh = rb[_NBUF]
    gsems = rb[_NBUF + 1:2 * _NBUF + 1]
    if with_count:
      ones_v, zc_v, cnt_sh = rb[2 * _NBUF + 1:]
    cid = lax.axis_index("c")
    sid = lax.axis_index("s")
    wid = sid * _NC + cid
    row0 = sid * _RPT

    pltpu.sync_copy(e3.at[0, pl.ds(wid * _NCHUNK, _NCHUNK)], srcs_v)
    pltpu.sync_copy(e3.at[1, pl.ds(wid * _NCHUNK, _NCHUNK)], dsts_v)

    zlanes = 16 * 4 // jnp.dtype(dtype).itemsize
    def zfill(i, _):
      for c in range(D // zlanes):
        rows[0][i, pl.ds(c * zlanes, zlanes)] = jnp.zeros((zlanes,), dtype)
      return 0
    lax.fori_loop(0, _K, zfill, 0)
    for t in range(_RPT // _K):
      pltpu.sync_copy(rows[0], acc_sh.at[pl.ds(row0 + t * _K, _K)])
    rem = _RPT % _K
    if rem:
      pltpu.sync_copy(rows[0].at[pl.ds(0, rem)],
                      acc_sh.at[pl.ds(row0 + (_RPT // _K) * _K, rem)])

    if with_count:
      def ofill(i, _):
        ones_v[i, pl.ds(0, 16)] = jnp.ones((16,), jnp.float32)
        return 0
      lax.fori_loop(0, _K, ofill, 0)
      def zcfill(i, _):
        zc_v[i, pl.ds(0, 16)] = jnp.zeros((16,), jnp.float32)
        return 0
      lax.fori_loop(0, _RPT, zcfill, 0)
      pltpu.sync_copy(zc_v, cnt_sh.at[pl.ds(row0, _RPT)])

    plsc.subcore_barrier()

    for b in range(_NBUF):
      pltpu.async_copy(table.at[srcs_v.at[b]], rows[b], gsems[b])

    @pl.loop(0, _NCHUNK - _NBUF, step=_NBUF)
    def _steady(g):
      for b in range(_NBUF):
        j = g + b
        pltpu.make_async_copy(table.at[srcs_v.at[b]], rows[b], gsems[b]).wait()
        pltpu.sync_copy(rows[b], acc_sh.at[dsts_v.at[j]], add=True)
        if with_count:
          pltpu.sync_copy(ones_v, cnt_sh.at[dsts_v.at[j]], add=True)
        pltpu.async_copy(table.at[srcs_v.at[j + _NBUF]], rows[b], gsems[b])

    for b in range(_NBUF):
      j = _NCHUNK - _NBUF + b
      pltpu.make_async_copy(table.at[srcs_v.at[b]], rows[b], gsems[b]).wait()
      pltpu.sync_copy(rows[b], acc_sh.at[dsts_v.at[j]], add=True)
      if with_count:
        pltpu.sync_copy(ones_v, cnt_sh.at[dsts_v.at[j]], add=True)

    plsc.subcore_barrier()

    pltpu.sync_copy(acc_sh.at[pl.ds(row0, _RPT)],
                    out_p.at[cid, pl.ds(row0, _RPT)])
    if with_count:
      pltpu.sync_copy(cnt_sh.at[pl.ds(row0, _RPT)],
                      out_c.at[cid, pl.ds(row0, _RPT)])

  return pl.kernel(
      body, out_type=out_type, mesh=mesh, scratch_types=scratch,
      compiler_params=pltpu.CompilerParams(use_tc_tiling_on_sc=False))


_agg_c = _make_agg(128, with_count=True, _NBUF=4, dtype=jnp.bfloat16)
_agg_b = _make_agg(64, with_count=False, _NBUF=8, dtype=jnp.bfloat16)


_RB = 5056
_GRID = _NP // _RB


def _tc1_body(x_ref, wl_ref, wr_ref, a1_ref, r1_ref):
  x = x_ref[...]
  a1 = jnp.dot(x, wl_ref[...], preferred_element_type=jnp.float32)
  a1_ref[...] = a1.astype(jnp.bfloat16)
  r1 = jnp.dot(x, wr_ref[...], preferred_element_type=jnp.float32)
  r1_ref[...] = r1.astype(jnp.bfloat16)


_tc1 = pl.pallas_call(
    _tc1_body,
    grid=(_GRID,),
    in_specs=[
        pl.BlockSpec((_RB, _F), lambda i: (i, 0)),
        pl.BlockSpec((_F, _H), lambda i: (0, 0)),
        pl.BlockSpec((_F, _H), lambda i: (0, 0)),
    ],
    out_specs=[
        pl.BlockSpec((_RB, _H), lambda i: (i, 0)),
        pl.BlockSpec((_RB, _H), lambda i: (i, 0)),
    ],
    out_shape=[
        jax.ShapeDtypeStruct((_NP, _H), jnp.bfloat16),
        jax.ShapeDtypeStruct((_NP, _H), jnp.bfloat16),
    ],
)


def _tc2_body(p_ref, c_ref, r1_ref, b1_ref, wl_ref, wr_ref,
              a2_ref, r2_ref):
  s1 = p_ref[0].astype(jnp.float32) + p_ref[1].astype(jnp.float32)
  cnt = jnp.max(c_ref[0] + c_ref[1], axis=1, keepdims=True)
  mean = s1 / jnp.maximum(cnt, 1.0)
  h = jnp.maximum(mean + r1_ref[...].astype(jnp.float32) + b1_ref[...], 0.0)
  a2 = jnp.dot(h, wl_ref[...], preferred_element_type=jnp.float32)
  a2_ref[...] = a2.astype(jnp.bfloat16)
  r2 = jnp.dot(h, wr_ref[...], preferred_element_type=jnp.float32)
  r2_ref[...] = r2.astype(jnp.bfloat16)


_tc2 = pl.pallas_call(
    _tc2_body,
    grid=(_GRID,),
    in_specs=[
        pl.BlockSpec((_NC, _RB, _H), lambda i: (0, i, 0)),
        pl.BlockSpec((_NC, _RB, 16), lambda i: (0, i, 0)),
        pl.BlockSpec((_RB, _H), lambda i: (i, 0)),
        pl.BlockSpec((1, _H), lambda i: (0, 0)),
        pl.BlockSpec((_H, _C), lambda i: (0, 0)),
        pl.BlockSpec((_H, _C), lambda i: (0, 0)),
    ],
    out_specs=[
        pl.BlockSpec((_RB, _C), lambda i: (i, 0)),
        pl.BlockSpec((_RB, _C), lambda i: (i, 0)),
    ],
    out_shape=[
        jax.ShapeDtypeStruct((_NP, _C), jnp.bfloat16),
        jax.ShapeDtypeStruct((_NP, _C), jnp.bfloat16),
    ],
)


def _tc3_body(q_ref, c_ref, r2_ref, b2_ref, out_ref):
  s2 = q_ref[0].astype(jnp.float32) + q_ref[1].astype(jnp.float32)
  cnt = jnp.max(c_ref[0] + c_ref[1], axis=1, keepdims=True)
  o = (s2 / jnp.maximum(cnt, 1.0) + r2_ref[...].astype(jnp.float32)
       + b2_ref[...])
  m = jnp.max(o, axis=1, keepdims=True)
  lse = jnp.log(jnp.sum(jnp.exp(o - m), axis=1, keepdims=True)) + m
  out_ref[...] = o - lse


_tc3 = pl.pallas_call(
    _tc3_body,
    grid=(_GRID,),
    in_specs=[
        pl.BlockSpec((_NC, _RB, _C), lambda i: (0, i, 0)),
        pl.BlockSpec((_NC, _RB, 16), lambda i: (0, i, 0)),
        pl.BlockSpec((_RB, _C), lambda i: (i, 0)),
        pl.BlockSpec((1, _C), lambda i: (0, 0)),
    ],
    out_specs=pl.BlockSpec((_RB, _C), lambda i: (i, 0)),
    out_shape=jax.ShapeDtypeStruct((_NP, _C), jnp.float32),
)


@jax.jit
def kernel(x, edge_index, W1_l, W1_r, b1, W2_l, W2_r, b2):
  e3 = edge_index.reshape(2, _E // _K, _K)
  x_p = jnp.pad(x, ((0, _NP - _N), (0, 0)))
  a1, r1 = _tc1(x_p, W1_l, W1_r)
  p, c = _agg_c(a1, e3)
  a2, r2 = _tc2(p, c, r1, b1.reshape(1, _H), W2_l, W2_r)
  (q,) = _agg_b(a2, e3)
  out = _tc3(q, c, r2, b2.reshape(1, _C))
  return out[:_N]
